# Initial kernel scaffold; baseline (speedup 1.0000x reference)
import jax
import jax.numpy as jnp
from jax.experimental import pallas as pl


def kernel(pred, target):
    raise NotImplementedError("write your pallas kernel here")


# speedup vs baseline: 2.6240x; 2.6240x over previous
import functools

import jax
import jax.numpy as jnp
from jax.experimental import pallas as pl
from jax.experimental.pallas import tpu as pltpu

NUM_CLASSES_K = 100000
SMOOTHING_K = 0.1
CONFIDENCE_K = 1.0 - SMOOTHING_K
EPS_K = SMOOTHING_K / (NUM_CLASSES_K - 1)

ROWS = 1024
COL_CHUNK = 2048


def _loss_kernel(pred_ref, tgt_ref, out_ref, m_ref, s_ref, t_ref, g_ref,
                 *, num_chunks, num_cols):
    j = pl.program_id(0)

    @pl.when(j == 0)
    def _init():
        m_ref[...] = jnp.full_like(m_ref, -jnp.inf)
        s_ref[...] = jnp.zeros_like(s_ref)
        t_ref[...] = jnp.zeros_like(t_ref)
        g_ref[...] = jnp.zeros_like(g_ref)

    x = pred_ref[...]
    col = jax.lax.broadcasted_iota(jnp.int32, x.shape, 1) + j * COL_CHUNK
    valid = col < num_cols

    xm = jnp.where(valid, x, -jnp.inf)
    m_old = m_ref[...]
    chunk_max = jnp.max(xm, axis=1, keepdims=True)
    m_new = jnp.maximum(m_old, chunk_max)
    s_ref[...] = s_ref[...] * jnp.exp(m_old - m_new) + jnp.sum(
        jnp.exp(xm - m_new), axis=1, keepdims=True)
    m_ref[...] = m_new

    t_ref[...] = t_ref[...] + jnp.sum(
        jnp.where(valid, x, 0.0), axis=1, keepdims=True)

    hit = col == tgt_ref[...]
    g_ref[...] = g_ref[...] + jnp.sum(
        jnp.where(hit, x, 0.0), axis=1, keepdims=True)

    @pl.when(j == num_chunks - 1)
    def _finalize():
        lse = m_ref[...] + jnp.log(s_ref[...])
        row_loss = (-EPS_K * (t_ref[...] - num_cols * lse)
                    - (CONFIDENCE_K - EPS_K) * (g_ref[...] - lse))
        out_ref[...] = jnp.sum(row_loss).reshape(1, 1) / ROWS


def kernel(pred, target):
    rows, num_cols = pred.shape
    num_chunks = pl.cdiv(num_cols, COL_CHUNK)
    tgt2d = target.astype(jnp.int32).reshape(rows, 1)

    out = pl.pallas_call(
        functools.partial(_loss_kernel, num_chunks=num_chunks,
                          num_cols=num_cols),
        grid=(num_chunks,),
        in_specs=[
            pl.BlockSpec((rows, COL_CHUNK), lambda j: (0, j)),
            pl.BlockSpec((rows, 1), lambda j: (0, 0)),
        ],
        out_specs=pl.BlockSpec((1, 1), lambda j: (0, 0)),
        out_shape=jax.ShapeDtypeStruct((1, 1), jnp.float32),
        scratch_shapes=[
            pltpu.VMEM((rows, 1), jnp.float32),
            pltpu.VMEM((rows, 1), jnp.float32),
            pltpu.VMEM((rows, 1), jnp.float32),
            pltpu.VMEM((rows, 1), jnp.float32),
        ],
    )(pred, tgt2d)
    return out[0, 0]
